# baseline (speedup 1.0000x reference)
import functools

import jax
import jax.numpy as jnp
from jax import lax
from jax.experimental import pallas as pl
from jax.experimental.pallas import tpu as pltpu
from jax.experimental.pallas import tpu_sc as plsc


BT = 128

_SC_INFO = plsc.get_sparse_core_info()
_NW = _SC_INFO.num_cores * _SC_INFO.num_subcores


def _sc_dispatch(inputs, ids, E, P, nbp):
    B, D = inputs.shape
    NS = _SC_INFO.num_subcores
    CH = B // NS
    HF = CH // _SC_INFO.num_cores
    NV = CH // 16
    nbm = ((nbp + 15) // 16) * 16
    NBH = nbm // 16
    mesh = plsc.VectorSubcoreMesh(core_axis_name="c", subcore_axis_name="s")

    @functools.partial(
        pl.kernel, mesh=mesh,
        out_type=[
            jax.ShapeDtypeStruct((P, D), jnp.float32),
            jax.ShapeDtypeStruct((B,), jnp.int32),
            jax.ShapeDtypeStruct((nbm,), jnp.int32),
            jax.ShapeDtypeStruct((nbm,), jnp.int32),
        ],
        scratch_types=[
            pltpu.VMEM((CH,), jnp.int32),
            pltpu.VMEM((16,), jnp.int32),
            pltpu.VMEM((NS * 16,), jnp.int32),
            pltpu.VMEM_SHARED((NS * 16,), jnp.int32),
            pltpu.VMEM((HF,), jnp.int32),
            pltpu.VMEM((HF, D), jnp.float32),
            pltpu.VMEM((nbm,), jnp.int32),
            pltpu.VMEM((nbm,), jnp.int32),
            pltpu.SemaphoreType.DMA,
        ],
    )
    def k(x_hbm, ids_hbm, xp_hbm, pos_hbm, be_hbm, bi_hbm,
          idsv, cntv, cnt_all, cnt_sh, posv, rows, bev, biv, sem):
        c = lax.axis_index("c")
        s = lax.axis_index("s")
        lane = lax.iota(jnp.int32, 16)
        idx_last = lane * 0 + 15

        def ieq(a, b):
            return 1 - jnp.minimum(jnp.abs(a - b), 1)

        def ige(a, b):
            return jnp.minimum(jnp.maximum(a - b + 1, 0), 1)

        def vsum(v):
            for kk in (8, 4, 2, 1):
                v = v + v[jnp.bitwise_xor(lane, kk)]
            return v

        def vscan(v):
            for kk in (1, 2, 4, 8):
                v = v + v[jnp.bitwise_and(lane - kk, 15)] * ige(lane, kk)
            return v

        pltpu.sync_copy(ids_hbm.at[pl.ds(s * CH, CH)], idsv)
        cnt = None
        cnt_half = None
        for e in range(E):
            acc = None
            acc_half = None
            for j in range(NV):
                m = ieq(idsv[pl.ds(16 * j, 16)], e)
                acc = m if acc is None else acc + m
                if j == NV // 2 - 1:
                    acc_half = acc
            sel = ieq(lane, e) * vsum(acc)
            sel_half = ieq(lane, e) * vsum(acc_half)
            cnt = sel if cnt is None else cnt + sel
            cnt_half = sel_half if cnt_half is None else cnt_half + sel_half
        cntv[...] = cnt
        pltpu.sync_copy(cntv, cnt_sh.at[pl.ds(16 * s, 16)])
        plsc.subcore_barrier()
        pltpu.sync_copy(cnt_sh, cnt_all)

        total = None
        prefix = None
        for j in range(NS):
            row = cnt_all[pl.ds(16 * j, 16)]
            pterm = row * jnp.minimum(jnp.maximum(s - j, 0), 1)
            prefix = pterm if prefix is None else prefix + pterm
            total = row if total is None else total + row
        padded = jnp.bitwise_and(total + (BT - 1), -BT)
        pad_end = vscan(padded)
        base_vec = (pad_end - padded) + prefix + cnt_half * c

        my_base = s * CH + c * HF
        pvs = [None] * (HF // 16)
        for e in range(E):
            run = base_vec[lane * 0 + e]
            for j in range(HF // 16):
                m = ieq(idsv[pl.ds(c * HF + 16 * j, 16)], e)
                occ = vscan(m)
                term = m * (run + occ - 1)
                pvs[j] = term if pvs[j] is None else pvs[j] + term
                run = run + occ[idx_last]
        for j in range(HF // 16):
            posv[pl.ds(16 * j, 16)] = pvs[j]
        pltpu.sync_copy(posv, pos_hbm.at[pl.ds(my_base, HF)])

        pltpu.sync_copy(x_hbm.at[pl.ds(my_base, HF)], rows)
        pltpu.async_copy(rows, xp_hbm.at[posv], sem).wait()

        @pl.when(c + s == 0)
        def _():
            nvalid = jnp.right_shift(pad_end[idx_last], BT.bit_length() - 1)
            last = nvalid - 1
            bes = []
            for h in range(NBH):
                blk = lane + 16 * h
                acc = None
                for e in range(E):
                    pe = pad_end[lane * 0 + e]
                    t = ige(blk * BT, pe)
                    acc = t if acc is None else acc + t
                bes.append(jnp.minimum(acc, E - 1))
            be_last = None
            for h in range(NBH):
                t = ieq(lane, last - 16 * h) * bes[h]
                be_last = t if be_last is None else be_last + t
            be_last = vsum(be_last)
            for h in range(NBH):
                blk = lane + 16 * h
                vi = 1 - ige(blk, nvalid)
                bev[pl.ds(16 * h, 16)] = bes[h] * vi + be_last * (1 - vi)
                biv[pl.ds(16 * h, 16)] = jnp.minimum(blk, last)
            pltpu.sync_copy(bev, be_hbm)
            pltpu.sync_copy(biv, bi_hbm)

    return k(inputs, ids)


def _sc_gather2(tab_a, tab_b, idx):
    n = idx.shape[0]
    da, db = tab_a.shape[1], tab_b.shape[1]
    bpw = n // _NW
    mesh = plsc.VectorSubcoreMesh(core_axis_name="c", subcore_axis_name="s")

    @functools.partial(
        pl.kernel, mesh=mesh,
        out_type=[
            jax.ShapeDtypeStruct((n, da), tab_a.dtype),
            jax.ShapeDtypeStruct((n, db), tab_b.dtype),
        ],
        scratch_types=[
            pltpu.VMEM((bpw,), jnp.int32),
            pltpu.VMEM((bpw, da), tab_a.dtype),
            pltpu.VMEM((bpw, db), tab_b.dtype),
            pltpu.SemaphoreType.DMA,
        ],
    )
    def k(a_hbm, b_hbm, idx_hbm, out_a, out_b, idx_v, rows_a, rows_b, sem):
        wid = lax.axis_index("s") * _SC_INFO.num_cores + lax.axis_index("c")
        base = wid * bpw
        pltpu.sync_copy(idx_hbm.at[pl.ds(base, bpw)], idx_v)
        cp_a = pltpu.async_copy(a_hbm.at[idx_v], rows_a, sem)
        cp_b = pltpu.async_copy(b_hbm.at[idx_v], rows_b, sem)
        cp_a.wait()
        cp_b.wait()
        pltpu.sync_copy(rows_a, out_a.at[pl.ds(base, bpw)])
        pltpu.sync_copy(rows_b, out_b.at[pl.ds(base, bpw)])

    return k(tab_a, tab_b, idx)


def _tc_body(be_ref, bi_ref, x_ref, w1a, b1a, w2a, b2a, w1c, b1c, w2c, b2c,
             vw, vb, y_ref, v_ref):
    f32, bf16 = jnp.float32, jnp.bfloat16
    e = be_ref[pl.program_id(0)]
    x = x_ref[...].astype(bf16)
    h = jnp.tanh(jnp.dot(x, w1a[0].astype(bf16), preferred_element_type=f32)
                 + b1a[e]).astype(bf16)
    ha = jnp.tanh(jnp.dot(h, w2a[0].astype(bf16), preferred_element_type=f32)
                  + b2a[e])
    y_ref[...] = ha
    g = jnp.tanh(jnp.dot(x, w1c[0].astype(bf16), preferred_element_type=f32)
                 + b1c[e]).astype(bf16)
    hc = jnp.tanh(jnp.dot(g, w2c[0].astype(bf16), preferred_element_type=f32)
                  + b2c[e]).astype(bf16)
    v = jnp.dot(hc, vw[e].astype(bf16), preferred_element_type=f32) + vb[e]
    v_ref[...] = jnp.broadcast_to(v, v_ref.shape)


def _expert_mlp(x_padded, block_expert, block_src, AW1, Ab1, AW2, Ab2,
                CW1, Cb1, CW2, Cb2, VW, Vb):
    P, D = x_padded.shape
    E, _, H = AW1.shape
    nbp = P // BT
    grid_spec = pltpu.PrefetchScalarGridSpec(
        num_scalar_prefetch=2,
        grid=(nbp,),
        in_specs=[
            pl.BlockSpec((BT, D), lambda i, be, bi: (bi[i], 0)),
            pl.BlockSpec((1, D, H), lambda i, be, bi: (be[i], 0, 0)),
            pl.BlockSpec((E, H), lambda i, be, bi: (0, 0)),
            pl.BlockSpec((1, H, H), lambda i, be, bi: (be[i], 0, 0)),
            pl.BlockSpec((E, H), lambda i, be, bi: (0, 0)),
            pl.BlockSpec((1, D, H), lambda i, be, bi: (be[i], 0, 0)),
            pl.BlockSpec((E, H), lambda i, be, bi: (0, 0)),
            pl.BlockSpec((1, H, H), lambda i, be, bi: (be[i], 0, 0)),
            pl.BlockSpec((E, H), lambda i, be, bi: (0, 0)),
            pl.BlockSpec((E, H, 1), lambda i, be, bi: (0, 0, 0)),
            pl.BlockSpec((E, 1), lambda i, be, bi: (0, 0)),
        ],
        out_specs=[
            pl.BlockSpec((BT, H), lambda i, be, bi: (bi[i], 0)),
            pl.BlockSpec((BT, 128), lambda i, be, bi: (bi[i], 0)),
        ],
    )
    return pl.pallas_call(
        _tc_body,
        grid_spec=grid_spec,
        out_shape=[
            jax.ShapeDtypeStruct((P, H), jnp.float32),
            jax.ShapeDtypeStruct((P, 128), jnp.float32),
        ],
    )(block_expert, block_src, x_padded,
      AW1, Ab1, AW2, Ab2, CW1, Cb1, CW2, Cb2, VW, Vb)


def kernel(inputs, rnn_hxs, masks, controller_ids, AW1, Ab1, AW2, Ab2,
           CW1, Cb1, CW2, Cb2, VW, Vb):
    B, D = inputs.shape
    E, _, H = AW1.shape
    P = B + E * BT
    nbp = P // BT

    ids = controller_ids.astype(jnp.int32)
    x_padded, pos, block_expert, block_src = _sc_dispatch(inputs, ids, E, P,
                                                          nbp)

    y_padded, v_padded = _expert_mlp(
        x_padded, block_expert, block_src,
        AW1, Ab1, AW2, Ab2, CW1, Cb1, CW2, Cb2, VW, Vb)

    actor_features, value = _sc_gather2(y_padded, v_padded, pos)
    return value[:, :1], actor_features, rnn_hxs
